# baseline (speedup 1.0000x reference)
import functools

import jax
import jax.numpy as jnp
from jax import lax
from jax.experimental import pallas as pl
from jax.experimental.pallas import tpu as pltpu
from jax.experimental.pallas import tpu_sc as plsc

DIM = 64
BATCH = 16384
NC, NS, L = 2, 16, 16
NW = NC * NS
B_PER_W = BATCH // NW
C = 128
CHUNK_SIZES = (128, 128, 128, 128)
CHUNK_OFFS = (0, 128, 256, 384)
N_CHUNK = len(CHUNK_SIZES)


def _trig_body(rel_ref, cos_ref, sin_ref):
    x = rel_ref[...]
    cos_ref[...] = jnp.cos(x)
    sin_ref[...] = jnp.sin(x)


def _trig(rel):
    cos_t, sin_t = pl.pallas_call(
        _trig_body,
        out_shape=(
            jax.ShapeDtypeStruct(rel.shape, rel.dtype),
            jax.ShapeDtypeStruct(rel.shape, rel.dtype),
        ),
    )(rel)
    return jnp.concatenate([cos_t, sin_t], axis=-1)


_mesh = plsc.VectorSubcoreMesh(
    core_axis_name="c", subcore_axis_name="s", num_cores=NC, num_subcores=NS
)


@functools.partial(
    pl.kernel,
    out_type=jax.ShapeDtypeStruct((BATCH,), jnp.float32),
    mesh=_mesh,
    scratch_types=[
        pltpu.VMEM((B_PER_W,), jnp.int32),
        pltpu.VMEM((B_PER_W,), jnp.int32),
        pltpu.VMEM((B_PER_W,), jnp.int32),
        pltpu.VMEM((2, C, 2 * DIM), jnp.float32),
        pltpu.VMEM((2, C, 2 * DIM), jnp.float32),
        pltpu.VMEM((2, C, 2 * DIM), jnp.float32),
        pltpu.VMEM((2, C), jnp.float32),
        pltpu.SemaphoreType.DMA,
        pltpu.SemaphoreType.DMA,
        pltpu.SemaphoreType.DMA,
        pltpu.SemaphoreType.DMA,
    ],
)
def _sc_rotate(h_hbm, r_hbm, t_hbm, ent_hbm, cs_hbm, out_hbm,
               idx_h, idx_t, idx_r, hbuf, tbuf, csbuf, obuf,
               sem0, sem1, osem0, osem1):
    wid = lax.axis_index("s") * NC + lax.axis_index("c")
    base = wid * B_PER_W
    sems = (sem0, sem1)
    osems = (osem0, osem1)

    icps = (
        pltpu.async_copy(h_hbm.at[pl.ds(base, B_PER_W)], idx_h, sem0),
        pltpu.async_copy(t_hbm.at[pl.ds(base, B_PER_W)], idx_t, sem0),
        pltpu.async_copy(r_hbm.at[pl.ds(base, B_PER_W)], idx_r, sem0),
    )
    for cp in icps:
        cp.wait()

    def fire(ci, s):
        sem = sems[s]
        off, n = CHUNK_OFFS[ci], CHUNK_SIZES[ci]
        sl = pl.ds(off, n)
        dst = pl.ds(0, n)
        return (
            pltpu.async_copy(ent_hbm.at[idx_h.at[sl]], hbuf.at[s].at[dst], sem),
            pltpu.async_copy(ent_hbm.at[idx_t.at[sl]], tbuf.at[s].at[dst], sem),
            pltpu.async_copy(cs_hbm.at[idx_r.at[sl]], csbuf.at[s].at[dst], sem),
        )

    lanes = lax.iota(jnp.int32, L)
    perms = [jnp.bitwise_xor(lanes, s) for s in (8, 4, 2, 1)]

    def compute(s, n):
        hb, tb, cb, ob = hbuf.at[s], tbuf.at[s], csbuf.at[s], obuf.at[s]

        def block_body(b, carry2):
            res = jnp.zeros((L,), jnp.float32)
            for i2 in range(L):
                i = b * L + i2
                acc = jnp.zeros((L,), jnp.float32)
                for j in range(DIM // L):
                    lo = pl.ds(j * L, L)
                    hi_sl = pl.ds(DIM + j * L, L)
                    hr = hb[i, lo]
                    hi = hb[i, hi_sl]
                    tr = tb[i, lo]
                    ti = tb[i, hi_sl]
                    cz = cb[i, lo]
                    sz = cb[i, hi_sl]
                    rr = hr * cz - hi * sz - tr
                    ri = hr * sz + hi * cz - ti
                    acc = acc + jnp.abs(rr) + jnp.abs(ri)
                for p in perms:
                    acc = acc + jnp.take(acc, p)
                res = jnp.where(lanes == i2, -acc, res)
            ob[pl.ds(b * L, L)] = res
            return carry2

        lax.fori_loop(0, n // L, block_body, 0)

    cps = fire(0, 0)
    ocps = [None, None]
    for ci in range(N_CHUNK):
        s = ci % 2
        off, n = CHUNK_OFFS[ci], CHUNK_SIZES[ci]
        for cp in cps:
            cp.wait()
        if ci + 1 < N_CHUNK:
            cps = fire(ci + 1, 1 - s)
        if ocps[s] is not None:
            ocps[s].wait()
        compute(s, n)
        ocps[s] = pltpu.async_copy(
            obuf.at[s].at[pl.ds(0, n)],
            out_hbm.at[pl.ds(base + off, n)],
            osems[s],
        )
    ocps[0].wait()
    ocps[1].wait()


def kernel(h, r, t, ent, rel):
    cs = _trig(rel)
    return _sc_rotate(h, r, t, ent, cs)


# speedup vs baseline: 1.0864x; 1.0864x over previous
import functools

import jax
import jax.numpy as jnp
from jax import lax
from jax.experimental import pallas as pl
from jax.experimental.pallas import tpu as pltpu
from jax.experimental.pallas import tpu_sc as plsc

DIM = 64
BATCH = 16384
NC, NS, L = 2, 16, 16
NW = NC * NS
B_PER_W = BATCH // NW
C = 128
N_CHUNK = B_PER_W // C


def _trig_body(rel_ref, cs_ref):
    x = rel_ref[...]
    cs_ref[...] = jnp.concatenate([jnp.cos(x), jnp.sin(x)], axis=-1)


def _trig(rel):
    return pl.pallas_call(
        _trig_body,
        out_shape=jax.ShapeDtypeStruct((rel.shape[0], 2 * rel.shape[1]), rel.dtype),
    )(rel)


_mesh = plsc.VectorSubcoreMesh(
    core_axis_name="c", subcore_axis_name="s", num_cores=NC, num_subcores=NS
)


@functools.partial(
    pl.kernel,
    out_type=jax.ShapeDtypeStruct((BATCH,), jnp.float32),
    mesh=_mesh,
    scratch_types=[
        pltpu.VMEM((B_PER_W,), jnp.int32),
        pltpu.VMEM((B_PER_W,), jnp.int32),
        pltpu.VMEM((B_PER_W,), jnp.int32),
        pltpu.VMEM((2, C, 2 * DIM), jnp.float32),
        pltpu.VMEM((2, C, 2 * DIM), jnp.float32),
        pltpu.VMEM((2, C, 2 * DIM), jnp.float32),
        pltpu.VMEM((2, C), jnp.float32),
        pltpu.SemaphoreType.DMA,
        pltpu.SemaphoreType.DMA,
        pltpu.SemaphoreType.DMA,
        pltpu.SemaphoreType.DMA,
    ],
)
def _sc_rotate(h_hbm, r_hbm, t_hbm, ent_hbm, cs_hbm, out_hbm,
               idx_h, idx_t, idx_r, hbuf, tbuf, csbuf, obuf,
               sem0, sem1, osem0, osem1):
    wid = lax.axis_index("s") * NC + lax.axis_index("c")
    base = wid * B_PER_W
    sems = (sem0, sem1)
    osems = (osem0, osem1)

    icps = (
        pltpu.async_copy(h_hbm.at[pl.ds(base, B_PER_W)], idx_h, sem0),
        pltpu.async_copy(t_hbm.at[pl.ds(base, B_PER_W)], idx_t, sem0),
        pltpu.async_copy(r_hbm.at[pl.ds(base, B_PER_W)], idx_r, sem0),
    )
    for cp in icps:
        cp.wait()

    def gather_descs(ci, s):
        sl = pl.ds(ci * C, C)
        return (
            (ent_hbm.at[idx_h.at[sl]], hbuf.at[s], sems[s]),
            (ent_hbm.at[idx_t.at[sl]], tbuf.at[s], sems[s]),
            (cs_hbm.at[idx_r.at[sl]], csbuf.at[s], sems[s]),
        )

    def fire(ci, s):
        for d in gather_descs(ci, s):
            pltpu.async_copy(*d)

    def wait_gathers(ci, s):
        for d in gather_descs(ci, s):
            pltpu.make_async_copy(*d).wait()

    lanes = lax.iota(jnp.int32, L)
    perms = [jnp.bitwise_xor(lanes, s) for s in (8, 4, 2, 1)]

    def compute(s):
        hb, tb, cb, ob = hbuf.at[s], tbuf.at[s], csbuf.at[s], obuf.at[s]

        def block_body(b, carry2):
            res = jnp.zeros((L,), jnp.float32)
            for i2 in range(L):
                i = b * L + i2
                acc = jnp.zeros((L,), jnp.float32)
                for j in range(DIM // L):
                    lo = pl.ds(j * L, L)
                    hi_sl = pl.ds(DIM + j * L, L)
                    hr = hb[i, lo]
                    hi = hb[i, hi_sl]
                    tr = tb[i, lo]
                    ti = tb[i, hi_sl]
                    cz = cb[i, lo]
                    sz = cb[i, hi_sl]
                    rr = hr * cz - hi * sz - tr
                    ri = hr * sz + hi * cz - ti
                    acc = acc + jnp.abs(rr) + jnp.abs(ri)
                for p in perms:
                    acc = acc + jnp.take(acc, p)
                res = jnp.where(lanes == i2, -acc, res)
            ob[pl.ds(b * L, L)] = res
            return carry2

        lax.fori_loop(0, C // L, block_body, 0)

    fire(0, 0)

    def pair_body(g, carry):
        for b in (0, 1):
            ci = 2 * g + b
            wait_gathers(ci, b)

            @pl.when(ci + 1 < N_CHUNK)
            def _():
                fire(ci + 1, 1 - b)

            @pl.when(ci >= 2)
            def _():
                pltpu.make_async_copy(
                    obuf.at[b],
                    out_hbm.at[pl.ds(base + (ci - 2) * C, C)],
                    osems[b],
                ).wait()

            compute(b)
            pltpu.async_copy(
                obuf.at[b], out_hbm.at[pl.ds(base + ci * C, C)], osems[b]
            )
        return carry

    lax.fori_loop(0, N_CHUNK // 2, pair_body, 0)
    pltpu.make_async_copy(
        obuf.at[0], out_hbm.at[pl.ds(base + (N_CHUNK - 2) * C, C)], osems[0]
    ).wait()
    pltpu.make_async_copy(
        obuf.at[1], out_hbm.at[pl.ds(base + (N_CHUNK - 1) * C, C)], osems[1]
    ).wait()


def kernel(h, r, t, ent, rel):
    cs = _trig(rel)
    return _sc_rotate(h, r, t, ent, cs)
